# Initial kernel scaffold; baseline (speedup 1.0000x reference)
import jax
import jax.numpy as jnp
from jax.experimental import pallas as pl


def kernel(x, edge_index, edge_attr, params):
    raise NotImplementedError("write your pallas kernel here")


# speedup vs baseline: 1.8747x; 1.8747x over previous
import functools
import math

import numpy as np
import jax
import jax.numpy as jnp
from jax import lax
from jax.experimental import pallas as pl
from jax.experimental.pallas import tpu as pltpu
from jax.experimental.pallas import tpu_sc as plsc

NC = 2
NS = 16
CH = 200

_HI = lax.Precision.HIGHEST


def _dot(a, b):
    return jnp.dot(a, b, precision=_HI, preferred_element_type=jnp.float32)


def _gelu(x):
    return 0.5 * x * (1.0 + lax.erf(x * (1.0 / math.sqrt(2.0))))


def _mesh():
    return plsc.VectorSubcoreMesh(core_axis_name="c", subcore_axis_name="s",
                                  num_cores=NC, num_subcores=NS)


def _zero2d(ref, rows, width):
    zv = jnp.zeros((16,), jnp.float32)

    def bd(i, carry):
        for w in range(width // 16):
            ref[i, pl.ds(w * 16, 16)] = zv
        return carry

    lax.fori_loop(0, rows, bd, 0)


def _sc_gather16(table_pad, idx):
    e_tot = idx.shape[0]
    per_w = e_tot // (NC * NS)
    n_ch = per_w // CH

    @functools.partial(
        pl.kernel,
        out_type=jax.ShapeDtypeStruct((e_tot, 16), jnp.float32),
        mesh=_mesh(),
        scratch_types=[
            pltpu.VMEM((CH,), jnp.int32),
            pltpu.VMEM((CH, 128), jnp.float32),
            pltpu.VMEM((CH, 16), jnp.float32),
            pltpu.SemaphoreType.DMA,
        ],
    )
    def k(tab_hbm, idx_hbm, out_hbm, idx_v, rows_v, o16, sem):
        wid = lax.axis_index("s") * NC + lax.axis_index("c")
        base = wid * per_w

        def body(ch, carry):
            off = base + ch * CH
            pltpu.sync_copy(idx_hbm.at[pl.ds(off, CH)], idx_v)
            pltpu.async_copy(tab_hbm.at[idx_v], rows_v, sem).wait()

            def cp(e, carry2):
                o16[e, pl.ds(0, 16)] = rows_v[e, pl.ds(0, 16)]
                return carry2

            lax.fori_loop(0, CH, cp, 0)
            pltpu.sync_copy(o16, out_hbm.at[pl.ds(off, CH)])
            return carry

        lax.fori_loop(0, n_ch, body, 0)

    return k(table_pad, idx)


def _sc_scatter32(msgp, dstp, npad):
    e_tot = msgp.shape[0]
    ch = 128
    per_w = e_tot // (NC * NS)
    n_ch = per_w // ch
    rpt = npad // NS
    zr = 32
    nz = rpt // zr

    @functools.partial(
        pl.kernel,
        out_type=jax.ShapeDtypeStruct((NC, npad, 128), jnp.float32),
        mesh=_mesh(),
        scratch_types=[
            pltpu.VMEM((ch,), jnp.int32),
            pltpu.VMEM((ch, 32), jnp.float32),
            pltpu.VMEM((ch, 128), jnp.float32),
            pltpu.VMEM((zr, 128), jnp.float32),
            pltpu.VMEM_SHARED((npad, 128), jnp.float32),
        ],
    )
    def k(msg_hbm, dst_hbm, out_hbm, dst_v, m_v, st, zb, acc):
        c = lax.axis_index("c")
        s = lax.axis_index("s")
        wid = s * NC + c
        base = wid * per_w
        _zero2d(zb, zr, 128)
        _zero2d(st, ch, 128)
        for j in range(nz):
            pltpu.sync_copy(zb, acc.at[pl.ds(s * rpt + j * zr, zr)])
        plsc.subcore_barrier()

        def body(cb, carry):
            off = base + cb * ch
            pltpu.sync_copy(dst_hbm.at[pl.ds(off, ch)], dst_v)
            pltpu.sync_copy(msg_hbm.at[pl.ds(off, ch)], m_v)

            def cp(e, carry2):
                st[e, pl.ds(0, 16)] = m_v[e, pl.ds(0, 16)]
                st[e, pl.ds(16, 16)] = m_v[e, pl.ds(16, 16)]
                return carry2

            lax.fori_loop(0, ch, cp, 0)
            pltpu.sync_copy(st, acc.at[dst_v], add=True)
            return carry

        lax.fori_loop(0, n_ch, body, 0)
        plsc.subcore_barrier()
        for j in range(nz):
            r0 = s * rpt + j * zr
            pltpu.sync_copy(acc.at[pl.ds(r0, zr)], out_hbm.at[c, pl.ds(r0, zr)])

    return k(msgp, dstp)


def _sc_alpha(q0, q1, k0, k1, u, ea, src, dst):
    e_tot = src.shape[0]
    ch = 80
    per_t = e_tot // NS
    n_ch = per_t // ch

    @functools.partial(
        pl.kernel,
        out_type=jax.ShapeDtypeStruct((NC, NS, per_t, 16), jnp.float32),
        mesh=_mesh(),
        scratch_types=[
            pltpu.VMEM((ch,), jnp.int32),
            pltpu.VMEM((ch,), jnp.int32),
            pltpu.VMEM((ch, 128), jnp.float32),
            pltpu.VMEM((ch, 128), jnp.float32),
            pltpu.VMEM((ch, 16), jnp.float32),
            pltpu.VMEM((ch, 128), jnp.float32),
            pltpu.VMEM((ch, 16), jnp.float32),
            pltpu.SemaphoreType.DMA,
            pltpu.SemaphoreType.DMA,
        ],
    )
    def k(q0_h, q1_h, k0_h, k1_h, u_h, ea_h, src_h, dst_h, out_h,
          di, si, qr, kr, er, ur, ab, sem1, sem2):
        c = lax.axis_index("c")
        s = lax.axis_index("s")
        base = s * per_t

        @pl.when(c == 1)
        def _():
            _zero2d(er, ch, 16)
            _zero2d(ur, ch, 128)

        def body(cb, carry):
            off = base + cb * ch
            loc = cb * ch
            pltpu.sync_copy(dst_h.at[pl.ds(off, ch)], di)
            pltpu.sync_copy(src_h.at[pl.ds(off, ch)], si)

            @pl.when(c == 0)
            def _():
                cp1 = pltpu.async_copy(q0_h.at[di], qr, sem1)
                cp2 = pltpu.async_copy(k0_h.at[si], kr, sem2)
                cp1.wait()
                cp2.wait()
                cp3 = pltpu.async_copy(u_h.at[di], ur, sem1)
                cp3.wait()
                pltpu.sync_copy(ea_h.at[pl.ds(off, ch)], er)

            @pl.when(c == 1)
            def _():
                cp1 = pltpu.async_copy(q1_h.at[di], qr, sem1)
                cp2 = pltpu.async_copy(k1_h.at[si], kr, sem2)
                cp1.wait()
                cp2.wait()

            def edge(e, carry2):
                acc = qr[e, pl.ds(0, 16)] * kr[e, pl.ds(0, 16)]
                for r in range(1, 8):
                    acc = acc + (qr[e, pl.ds(r * 16, 16)]
                                 * kr[e, pl.ds(r * 16, 16)])
                acc = acc + er[e, pl.ds(0, 16)] * ur[e, pl.ds(0, 16)]
                ab[e, pl.ds(0, 16)] = acc
                return carry2

            lax.fori_loop(0, ch, edge, 0)
            pltpu.sync_copy(ab, out_h.at[c, s, pl.ds(loc, ch)])
            return carry

        lax.fori_loop(0, n_ch, body, 0)

    return k(q0, q1, k0, k1, u, ea, src, dst)


def _sc_aggregate(v0, v1, rows32, src, dst, npad):
    e_tot = src.shape[0]
    ch = 80
    per_t = e_tot // NS
    n_ch = per_t // ch
    rpt = npad // NS
    zr = 32
    nz = rpt // zr

    @functools.partial(
        pl.kernel,
        out_type=jax.ShapeDtypeStruct((NC, npad, 128), jnp.float32),
        mesh=_mesh(),
        scratch_types=[
            pltpu.VMEM((ch,), jnp.int32),
            pltpu.VMEM((ch,), jnp.int32),
            pltpu.VMEM((ch, 32), jnp.float32),
            pltpu.VMEM((ch, 128), jnp.float32),
            pltpu.VMEM((zr, 128), jnp.float32),
            pltpu.VMEM_SHARED((npad, 128), jnp.float32),
            pltpu.SemaphoreType.DMA,
        ],
    )
    def k(v0_h, v1_h, ew_h, src_h, dst_h, outv_h,
          si, di, eb, vr, zb, acc, sem):
        c = lax.axis_index("c")
        s = lax.axis_index("s")
        base = s * per_t
        _zero2d(zb, zr, 128)
        for j in range(nz):
            pltpu.sync_copy(zb, acc.at[pl.ds(s * rpt + j * zr, zr)])
        plsc.subcore_barrier()

        def body(cb, carry):
            off = base + cb * ch
            pltpu.sync_copy(src_h.at[pl.ds(off, ch)], si)
            pltpu.sync_copy(dst_h.at[pl.ds(off, ch)], di)
            pltpu.sync_copy(ew_h.at[pl.ds(off, ch)], eb)

            @pl.when(c == 0)
            def _():
                pltpu.async_copy(v0_h.at[si], vr, sem).wait()

            @pl.when(c == 1)
            def _():
                pltpu.async_copy(v1_h.at[si], vr, sem).wait()

            def edge(e, carry2):
                wrow = eb[e, pl.ds(16, 16)]
                ev = jnp.full((16,), wrow[0], jnp.float32)
                for r in range(8):
                    vr[e, pl.ds(r * 16, 16)] = vr[e, pl.ds(r * 16, 16)] * ev
                return carry2

            lax.fori_loop(0, ch, edge, 0)
            pltpu.sync_copy(vr, acc.at[di], add=True)
            return carry

        lax.fori_loop(0, n_ch, body, 0)
        plsc.subcore_barrier()
        for j in range(nz):
            r0 = s * rpt + j * zr
            pltpu.sync_copy(acc.at[pl.ds(r0, zr)], outv_h.at[c, pl.ds(r0, zr)])

    return k(v0, v1, rows32, src, dst)


def _full_spec(shape):
    nd = len(shape)
    return pl.BlockSpec(shape, lambda i: (0,) * nd)


def _tc_nnconv_edge(ea, xs, p, in_c, out_c, with_count):
    e_tot = ea.shape[0]
    be = 1000
    grid = e_tot // be
    r_mat = jnp.asarray(np.kron(np.eye(in_c), np.ones((1, out_c))),
                        jnp.float32)
    s_mat = jnp.asarray(np.kron(np.ones((in_c, 1)), np.eye(out_c)),
                        jnp.float32)
    out_w = 32 if with_count else out_c

    def body(ea_ref, xs_ref, w1, b1, w2, b2, r_ref, s_ref, out_ref):
        h = _gelu(_dot(ea_ref[...], w1[...]) + b1[...])
        wrow = _dot(h, w2[...]) + b2[...]
        xrep = _dot(xs_ref[...], r_ref[...])
        msg = _dot(xrep * wrow, s_ref[...])
        if with_count:
            pad = jnp.zeros((be, 32 - out_c - 1), jnp.float32)
            ones = jnp.ones((be, 1), jnp.float32)
            out_ref[...] = jnp.concatenate([msg, ones, pad], axis=1)
        else:
            out_ref[...] = msg

    b1 = p['mlp_b1'].reshape(1, -1)
    b2 = p['mlp_b2'].reshape(1, -1)
    return pl.pallas_call(
        body,
        grid=(grid,),
        in_specs=[
            pl.BlockSpec((be, 16), lambda i: (i, 0)),
            pl.BlockSpec((be, in_c), lambda i: (i, 0)),
            _full_spec(p['mlp_w1'].shape),
            _full_spec(b1.shape),
            _full_spec(p['mlp_w2'].shape),
            _full_spec(b2.shape),
            _full_spec(r_mat.shape),
            _full_spec(s_mat.shape),
        ],
        out_specs=pl.BlockSpec((be, out_w), lambda i: (i, 0)),
        out_shape=jax.ShapeDtypeStruct((e_tot, out_w), jnp.float32),
    )(ea, xs, p['mlp_w1'], b1, p['mlp_w2'], b2, r_mat, s_mat)


def _tc_combine1(agg, x, p, n):
    bn = 1000
    grid = n // bn

    def body(agg_ref, x_ref, rw, b, x1_ref):
        s0 = agg_ref[0]
        s1 = agg_ref[1]
        cnt = s0[:, 16:17] + s1[:, 16:17]
        mean = (s0[:, 0:16] + s1[:, 0:16]) / jnp.clip(cnt, 1.0, None)
        x1 = mean + _dot(x_ref[...], rw[...]) + b[...]
        pad = jnp.zeros((bn, 128 - 17), jnp.float32)
        x1_ref[...] = jnp.concatenate([x1, cnt, pad], axis=1)

    b = p['bias'].reshape(1, -1)
    return pl.pallas_call(
        body,
        grid=(grid,),
        in_specs=[
            pl.BlockSpec((2, bn, 128), lambda i: (0, i, 0)),
            pl.BlockSpec((bn, 16), lambda i: (i, 0)),
            _full_spec(p['root_w'].shape),
            _full_spec(b.shape),
        ],
        out_specs=pl.BlockSpec((bn, 128), lambda i: (i, 0)),
        out_shape=jax.ShapeDtypeStruct((n, 128), jnp.float32),
    )(agg, x, p['root_w'], b)


def _tc_combine2_proj(agg, x1pad, p, proj, n):
    bn = 1000
    grid = n // bn

    def body(agg_ref, x1p_ref, rw, b, pw, pb, out_ref):
        x1p = x1p_ref[...]
        cnt = x1p[:, 16:17]
        mean = (agg_ref[0][:, 0:32] + agg_ref[1][:, 0:32]) / jnp.clip(cnt, 1.0, None)
        x2 = mean + _dot(x1p[:, 0:16], rw[...]) + b[...]
        out_ref[...] = _dot(x2, pw[...]) + pb[...]

    b = p['bias'].reshape(1, -1)
    pb = proj['b'].reshape(1, -1)
    return pl.pallas_call(
        body,
        grid=(grid,),
        in_specs=[
            pl.BlockSpec((2, bn, 128), lambda i: (0, i, 0)),
            pl.BlockSpec((bn, 128), lambda i: (i, 0)),
            _full_spec(p['root_w'].shape),
            _full_spec(b.shape),
            _full_spec(proj['w'].shape),
            _full_spec(pb.shape),
        ],
        out_specs=pl.BlockSpec((bn, 256), lambda i: (i, 0)),
        out_shape=jax.ShapeDtypeStruct((n, 256), jnp.float32),
    )(agg, x1pad, p['root_w'], b, proj['w'], pb)


def _tc_qkv(x, p, n):
    bn = 1000
    grid = n // bn

    def body(x_ref, wq, bq, wk, bk, wv, bv, wet,
             q0_r, q1_r, k0_r, k1_r, v0_r, v1_r, u_r):
        xb = x_ref[...]
        q = _dot(xb, wq[...]) + bq[...]
        kk = _dot(xb, wk[...]) + bk[...]
        v = _dot(xb, wv[...]) + bv[...]
        q0_r[...] = q[:, 0:128]
        q1_r[...] = q[:, 128:256]
        k0_r[...] = kk[:, 0:128]
        k1_r[...] = kk[:, 128:256]
        v0_r[...] = v[:, 0:128]
        v1_r[...] = v[:, 128:256]
        u_r[...] = jnp.concatenate(
            [_dot(q, wet[...]), jnp.zeros((bn, 112), jnp.float32)], axis=1)

    bq = p['bq'].reshape(1, -1)
    bk = p['bk'].reshape(1, -1)
    bv = p['bv'].reshape(1, -1)
    wet = p['we'].T
    half = jax.ShapeDtypeStruct((n, 128), jnp.float32)
    return pl.pallas_call(
        body,
        grid=(grid,),
        in_specs=[
            pl.BlockSpec((bn, 256), lambda i: (i, 0)),
            _full_spec((256, 256)), _full_spec((1, 256)),
            _full_spec((256, 256)), _full_spec((1, 256)),
            _full_spec((256, 256)), _full_spec((1, 256)),
            _full_spec((256, 16)),
        ],
        out_specs=[pl.BlockSpec((bn, 128), lambda i: (i, 0))] * 7,
        out_shape=[half] * 7,
    )(x, p['wq'], bq, p['wk'], bk, p['wv'], bv, wet)


def _tc_pmax(ap, scale):
    e_tot = ap.shape[1]
    be = 8000
    grid = e_tot // be

    def body(ap_ref, m_ref):
        a = jnp.sum(ap_ref[0] + ap_ref[1], axis=-1) * scale
        m_ref[...] = jnp.full((8, 128), jnp.max(a), jnp.float32)

    return pl.pallas_call(
        body,
        grid=(grid,),
        in_specs=[pl.BlockSpec((2, be, 16), lambda i: (0, i, 0))],
        out_specs=pl.BlockSpec((8, 128), lambda i: (i, 0)),
        out_shape=jax.ShapeDtypeStruct((grid * 8, 128), jnp.float32),
    )(ap)


def _tc_exp_rows(ap, pmax, ea, scale):
    e_tot = ap.shape[1]
    be = 8000
    grid = e_tot // be

    def body(ap_ref, m_ref, ea_ref, out_ref):
        cmax = jnp.max(m_ref[...])
        a = jnp.sum(ap_ref[0] + ap_ref[1], axis=-1, keepdims=True) * scale
        ew = jnp.exp(a - cmax)
        pad = jnp.zeros((be, 15), jnp.float32)
        out_ref[...] = jnp.concatenate([ea_ref[...] * ew, ew, pad], axis=1)

    return pl.pallas_call(
        body,
        grid=(grid,),
        in_specs=[
            pl.BlockSpec((2, be, 16), lambda i: (0, i, 0)),
            _full_spec(pmax.shape),
            pl.BlockSpec((be, 16), lambda i: (i, 0)),
        ],
        out_specs=pl.BlockSpec((be, 32), lambda i: (i, 0)),
        out_shape=jax.ShapeDtypeStruct((e_tot, 32), jnp.float32),
    )(ap, pmax, ea)


def _tc_tf_combine(outv, a32, x, p, lnp, n):
    bn = 1000
    grid = n // bn

    def body(ov_ref, a32_ref, x_ref, we, ws, bs, g_ref, b_ref, out_ref):
        full = jnp.concatenate([ov_ref[0], ov_ref[1]], axis=1)
        a32 = a32_ref[0] + a32_ref[1]
        s = a32[:, 16:17]
        amat = a32[:, 0:16]
        attn = (full + _dot(amat, we[...])) / (s + 1e-16)
        y = attn + _dot(x_ref[...], ws[...]) + bs[...] + x_ref[...]
        m = jnp.mean(y, axis=1, keepdims=True)
        v = jnp.mean((y - m) ** 2, axis=1, keepdims=True)
        yn = (y - m) / jnp.sqrt(v + 1e-5) * g_ref[...] + b_ref[...]
        out_ref[...] = _gelu(yn)

    bs = p['bs'].reshape(1, -1)
    g = lnp['g'].reshape(1, -1)
    b = lnp['b'].reshape(1, -1)
    return pl.pallas_call(
        body,
        grid=(grid,),
        in_specs=[
            pl.BlockSpec((2, bn, 128), lambda i: (0, i, 0)),
            pl.BlockSpec((2, bn, 128), lambda i: (0, i, 0)),
            pl.BlockSpec((bn, 256), lambda i: (i, 0)),
            _full_spec((16, 256)),
            _full_spec((256, 256)),
            _full_spec((1, 256)),
            _full_spec((1, 256)),
            _full_spec((1, 256)),
        ],
        out_specs=pl.BlockSpec((bn, 256), lambda i: (i, 0)),
        out_shape=jax.ShapeDtypeStruct((n, 256), jnp.float32),
    )(outv, a32, x, p['we'], p['ws'], bs, g, b)


def _tc_pool_head(x, head, n):
    def body(x_ref, hw, hb, out_ref):
        pooled = jnp.mean(x_ref[...], axis=0, keepdims=True)
        out_ref[...] = _dot(pooled, hw[...]) + hb[...]

    hb = head['b'].reshape(1, 1)
    return pl.pallas_call(
        body,
        grid=(1,),
        in_specs=[
            _full_spec((n, 256)),
            _full_spec((256, 1)),
            _full_spec((1, 1)),
        ],
        out_specs=_full_spec((1, 1)),
        out_shape=jax.ShapeDtypeStruct((1, 1), jnp.float32),
    )(x, head['w'], hb)


def kernel(x, edge_index, edge_attr, params):
    n = x.shape[0]
    npad = ((n + 1023) // 1024) * 1024
    e_tot = edge_index.shape[1]
    src = edge_index[0]
    dst = edge_index[1]
    gnn = params['gnn']

    e_pad = ((e_tot + 4095) // 4096) * 4096 - e_tot
    eap = jnp.pad(edge_attr, ((0, e_pad), (0, 0)))
    dstp = jnp.pad(dst, (0, e_pad))

    xpad = jnp.pad(x, ((0, 0), (0, 112)))
    xs1 = _sc_gather16(xpad, src)
    msg1 = _tc_nnconv_edge(edge_attr, xs1, gnn[0], 16, 16, with_count=True)
    agg1 = _sc_scatter32(jnp.pad(msg1, ((0, e_pad), (0, 0))), dstp, npad)
    x1pad = _tc_combine1(agg1, x, gnn[0], n)

    xs2 = _sc_gather16(x1pad, src)
    msg2 = _tc_nnconv_edge(edge_attr, xs2, gnn[1], 16, 32, with_count=False)
    agg2 = _sc_scatter32(jnp.pad(msg2, ((0, e_pad), (0, 0))), dstp, npad)
    xp = _tc_combine2_proj(agg2, x1pad, gnn[1], params['proj'], n)

    for i in range(2):
        tfp = params['tf'][i]
        q0, q1, k0, k1, v0, v1, u = _tc_qkv(xp, tfp, n)
        ap = _sc_alpha(q0, q1, k0, k1, u, edge_attr, src, dst)
        ap2 = ap.reshape(2, e_tot, 16)
        pmax = _tc_pmax(ap2, 1.0 / 16.0)
        rows32 = _tc_exp_rows(ap2, pmax, edge_attr, 1.0 / 16.0)
        outv = _sc_aggregate(v0, v1, rows32, src, dst, npad)
        a32 = _sc_scatter32(jnp.pad(rows32, ((0, e_pad), (0, 0))), dstp, npad)
        xp = _tc_tf_combine(outv, a32, xp, tfp, params['ln'][i], n)

    return _tc_pool_head(xp, params['head'], n)


# speedup vs baseline: 2.0574x; 1.0975x over previous
import functools
import math

import numpy as np
import jax
import jax.numpy as jnp
from jax import lax
from jax.experimental import pallas as pl
from jax.experimental.pallas import tpu as pltpu
from jax.experimental.pallas import tpu_sc as plsc

NC = 2
NS = 16
CH = 200

_HI = lax.Precision.HIGHEST


def _dot(a, b):
    return jnp.dot(a, b, precision=_HI, preferred_element_type=jnp.float32)


def _gelu(x):
    return 0.5 * x * (1.0 + lax.erf(x * (1.0 / math.sqrt(2.0))))


def _mesh():
    return plsc.VectorSubcoreMesh(core_axis_name="c", subcore_axis_name="s",
                                  num_cores=NC, num_subcores=NS)


def _zero2d(ref, rows, width):
    zv = jnp.zeros((16,), jnp.float32)

    def bd(i, carry):
        for w in range(width // 16):
            ref[i, pl.ds(w * 16, 16)] = zv
        return carry

    lax.fori_loop(0, rows, bd, 0)


def _sc_gather16(table_pad, idx):
    e_tot = idx.shape[0]
    per_w = e_tot // (NC * NS)
    n_ch = per_w // CH

    @functools.partial(
        pl.kernel,
        out_type=jax.ShapeDtypeStruct((e_tot, 16), jnp.float32),
        mesh=_mesh(),
        scratch_types=[
            pltpu.VMEM((CH,), jnp.int32),
            pltpu.VMEM((CH, 128), jnp.float32),
            pltpu.VMEM((CH, 16), jnp.float32),
            pltpu.SemaphoreType.DMA,
        ],
    )
    def k(tab_hbm, idx_hbm, out_hbm, idx_v, rows_v, o16, sem):
        wid = lax.axis_index("s") * NC + lax.axis_index("c")
        base = wid * per_w

        def body(ch, carry):
            off = base + ch * CH
            pltpu.sync_copy(idx_hbm.at[pl.ds(off, CH)], idx_v)
            pltpu.async_copy(tab_hbm.at[idx_v], rows_v, sem).wait()

            def cp(e, carry2):
                o16[e, pl.ds(0, 16)] = rows_v[e, pl.ds(0, 16)]
                return carry2

            lax.fori_loop(0, CH, cp, 0)
            pltpu.sync_copy(o16, out_hbm.at[pl.ds(off, CH)])
            return carry

        lax.fori_loop(0, n_ch, body, 0)

    return k(table_pad, idx)


def _sc_scatter32(msgp, dstp, npad):
    e_tot = msgp.shape[0]
    ch = 128
    per_w = e_tot // (NC * NS)
    n_ch = per_w // ch
    rpt = npad // NS
    zr = 32
    nz = rpt // zr

    @functools.partial(
        pl.kernel,
        out_type=jax.ShapeDtypeStruct((NC, npad, 128), jnp.float32),
        mesh=_mesh(),
        scratch_types=[
            pltpu.VMEM((ch,), jnp.int32),
            pltpu.VMEM((ch, 32), jnp.float32),
            pltpu.VMEM((ch, 128), jnp.float32),
            pltpu.VMEM((zr, 128), jnp.float32),
            pltpu.VMEM_SHARED((npad, 128), jnp.float32),
        ],
    )
    def k(msg_hbm, dst_hbm, out_hbm, dst_v, m_v, st, zb, acc):
        c = lax.axis_index("c")
        s = lax.axis_index("s")
        wid = s * NC + c
        base = wid * per_w
        _zero2d(zb, zr, 128)
        _zero2d(st, ch, 128)
        for j in range(nz):
            pltpu.sync_copy(zb, acc.at[pl.ds(s * rpt + j * zr, zr)])
        plsc.subcore_barrier()

        def body(cb, carry):
            off = base + cb * ch
            pltpu.sync_copy(dst_hbm.at[pl.ds(off, ch)], dst_v)
            pltpu.sync_copy(msg_hbm.at[pl.ds(off, ch)], m_v)

            def cp(e, carry2):
                st[e, pl.ds(0, 16)] = m_v[e, pl.ds(0, 16)]
                st[e, pl.ds(16, 16)] = m_v[e, pl.ds(16, 16)]
                return carry2

            lax.fori_loop(0, ch, cp, 0)
            pltpu.sync_copy(st, acc.at[dst_v], add=True)
            return carry

        lax.fori_loop(0, n_ch, body, 0)
        plsc.subcore_barrier()
        for j in range(nz):
            r0 = s * rpt + j * zr
            pltpu.sync_copy(acc.at[pl.ds(r0, zr)], out_hbm.at[c, pl.ds(r0, zr)])

    return k(msgp, dstp)


def _sc_alpha(q0, q1, k0, k1, u, ea, src, dst):
    e_tot = src.shape[0]
    ch = 200
    per_t = e_tot // NS
    n_ch = per_t // ch

    @functools.partial(
        pl.kernel,
        out_type=jax.ShapeDtypeStruct((NC, NS, per_t, 16), jnp.float32),
        mesh=_mesh(),
        scratch_types=[
            pltpu.VMEM((ch,), jnp.int32),
            pltpu.VMEM((ch,), jnp.int32),
            pltpu.VMEM((ch, 128), jnp.float32),
            pltpu.VMEM((ch, 128), jnp.float32),
            pltpu.VMEM((ch, 16), jnp.float32),
            pltpu.VMEM((ch, 128), jnp.float32),
            pltpu.VMEM((ch, 16), jnp.float32),
            pltpu.SemaphoreType.DMA,
            pltpu.SemaphoreType.DMA,
        ],
    )
    def k(q0_h, q1_h, k0_h, k1_h, u_h, ea_h, src_h, dst_h, out_h,
          di, si, qr, kr, er, ur, ab, sem1, sem2):
        c = lax.axis_index("c")
        s = lax.axis_index("s")
        base = s * per_t

        @pl.when(c == 1)
        def _():
            _zero2d(er, ch, 16)
            _zero2d(ur, ch, 128)

        def body(cb, carry):
            off = base + cb * ch
            loc = cb * ch
            pltpu.sync_copy(dst_h.at[pl.ds(off, ch)], di)
            pltpu.sync_copy(src_h.at[pl.ds(off, ch)], si)

            @pl.when(c == 0)
            def _():
                cp1 = pltpu.async_copy(q0_h.at[di], qr, sem1)
                cp2 = pltpu.async_copy(k0_h.at[si], kr, sem2)
                cp3 = pltpu.async_copy(u_h.at[di], ur, sem1)
                pltpu.sync_copy(ea_h.at[pl.ds(off, ch)], er)
                cp1.wait()
                cp2.wait()
                cp3.wait()

            @pl.when(c == 1)
            def _():
                cp1 = pltpu.async_copy(q1_h.at[di], qr, sem1)
                cp2 = pltpu.async_copy(k1_h.at[si], kr, sem2)
                cp1.wait()
                cp2.wait()

            def edge(e, carry2):
                acc = qr[e, pl.ds(0, 16)] * kr[e, pl.ds(0, 16)]
                for r in range(1, 8):
                    acc = acc + (qr[e, pl.ds(r * 16, 16)]
                                 * kr[e, pl.ds(r * 16, 16)])
                acc = acc + er[e, pl.ds(0, 16)] * ur[e, pl.ds(0, 16)]
                ab[e, pl.ds(0, 16)] = acc
                return carry2

            lax.fori_loop(0, ch, edge, 0)
            pltpu.sync_copy(ab, out_h.at[c, s, pl.ds(loc, ch)])
            return carry

        lax.fori_loop(0, n_ch, body, 0)

    return k(q0, q1, k0, k1, u, ea, src, dst)


def _sc_aggregate(v0, v1, rows32, src, dst, npad):
    e_tot = src.shape[0]
    ch = 80
    per_t = e_tot // NS
    n_ch = per_t // ch
    rpt = npad // NS
    zr = 32
    nz = rpt // zr

    @functools.partial(
        pl.kernel,
        out_type=jax.ShapeDtypeStruct((NC, npad, 128), jnp.float32),
        mesh=_mesh(),
        scratch_types=[
            pltpu.VMEM((ch,), jnp.int32),
            pltpu.VMEM((ch,), jnp.int32),
            pltpu.VMEM((ch, 32), jnp.float32),
            pltpu.VMEM((ch, 128), jnp.float32),
            pltpu.VMEM((zr, 128), jnp.float32),
            pltpu.VMEM_SHARED((npad, 128), jnp.float32),
            pltpu.SemaphoreType.DMA,
        ],
    )
    def k(v0_h, v1_h, ew_h, src_h, dst_h, outv_h,
          si, di, eb, vr, zb, acc, sem):
        c = lax.axis_index("c")
        s = lax.axis_index("s")
        base = s * per_t
        _zero2d(zb, zr, 128)
        for j in range(nz):
            pltpu.sync_copy(zb, acc.at[pl.ds(s * rpt + j * zr, zr)])
        plsc.subcore_barrier()

        def body(cb, carry):
            off = base + cb * ch
            pltpu.sync_copy(src_h.at[pl.ds(off, ch)], si)
            pltpu.sync_copy(dst_h.at[pl.ds(off, ch)], di)
            pltpu.sync_copy(ew_h.at[pl.ds(off, ch)], eb)

            @pl.when(c == 0)
            def _():
                pltpu.async_copy(v0_h.at[si], vr, sem).wait()

            @pl.when(c == 1)
            def _():
                pltpu.async_copy(v1_h.at[si], vr, sem).wait()

            def edge(e, carry2):
                wrow = eb[e, pl.ds(16, 16)]
                ev = jnp.full((16,), wrow[0], jnp.float32)
                for r in range(8):
                    vr[e, pl.ds(r * 16, 16)] = vr[e, pl.ds(r * 16, 16)] * ev
                return carry2

            lax.fori_loop(0, ch, edge, 0)
            pltpu.sync_copy(vr, acc.at[di], add=True)
            return carry

        lax.fori_loop(0, n_ch, body, 0)
        plsc.subcore_barrier()
        for j in range(nz):
            r0 = s * rpt + j * zr
            pltpu.sync_copy(acc.at[pl.ds(r0, zr)], outv_h.at[c, pl.ds(r0, zr)])

    return k(v0, v1, rows32, src, dst)


def _full_spec(shape):
    nd = len(shape)
    return pl.BlockSpec(shape, lambda i: (0,) * nd)


def _tc_nnconv_edge(ea, xs, p, in_c, out_c, with_count):
    e_tot = ea.shape[0]
    be = 1000
    grid = e_tot // be
    r_mat = jnp.asarray(np.kron(np.eye(in_c), np.ones((1, out_c))),
                        jnp.float32)
    s_mat = jnp.asarray(np.kron(np.ones((in_c, 1)), np.eye(out_c)),
                        jnp.float32)
    out_w = 32 if with_count else out_c

    def body(ea_ref, xs_ref, w1, b1, w2, b2, r_ref, s_ref, out_ref):
        h = _gelu(_dot(ea_ref[...], w1[...]) + b1[...])
        wrow = _dot(h, w2[...]) + b2[...]
        xrep = _dot(xs_ref[...], r_ref[...])
        msg = _dot(xrep * wrow, s_ref[...])
        if with_count:
            pad = jnp.zeros((be, 32 - out_c - 1), jnp.float32)
            ones = jnp.ones((be, 1), jnp.float32)
            out_ref[...] = jnp.concatenate([msg, ones, pad], axis=1)
        else:
            out_ref[...] = msg

    b1 = p['mlp_b1'].reshape(1, -1)
    b2 = p['mlp_b2'].reshape(1, -1)
    return pl.pallas_call(
        body,
        grid=(grid,),
        in_specs=[
            pl.BlockSpec((be, 16), lambda i: (i, 0)),
            pl.BlockSpec((be, in_c), lambda i: (i, 0)),
            _full_spec(p['mlp_w1'].shape),
            _full_spec(b1.shape),
            _full_spec(p['mlp_w2'].shape),
            _full_spec(b2.shape),
            _full_spec(r_mat.shape),
            _full_spec(s_mat.shape),
        ],
        out_specs=pl.BlockSpec((be, out_w), lambda i: (i, 0)),
        out_shape=jax.ShapeDtypeStruct((e_tot, out_w), jnp.float32),
    )(ea, xs, p['mlp_w1'], b1, p['mlp_w2'], b2, r_mat, s_mat)


def _tc_combine1(agg, x, p, n):
    bn = 1000
    grid = n // bn

    def body(agg_ref, x_ref, rw, b, x1_ref):
        s0 = agg_ref[0]
        s1 = agg_ref[1]
        cnt = s0[:, 16:17] + s1[:, 16:17]
        mean = (s0[:, 0:16] + s1[:, 0:16]) / jnp.clip(cnt, 1.0, None)
        x1 = mean + _dot(x_ref[...], rw[...]) + b[...]
        pad = jnp.zeros((bn, 128 - 17), jnp.float32)
        x1_ref[...] = jnp.concatenate([x1, cnt, pad], axis=1)

    b = p['bias'].reshape(1, -1)
    return pl.pallas_call(
        body,
        grid=(grid,),
        in_specs=[
            pl.BlockSpec((2, bn, 128), lambda i: (0, i, 0)),
            pl.BlockSpec((bn, 16), lambda i: (i, 0)),
            _full_spec(p['root_w'].shape),
            _full_spec(b.shape),
        ],
        out_specs=pl.BlockSpec((bn, 128), lambda i: (i, 0)),
        out_shape=jax.ShapeDtypeStruct((n, 128), jnp.float32),
    )(agg, x, p['root_w'], b)


def _tc_combine2_proj(agg, x1pad, p, proj, n):
    bn = 1000
    grid = n // bn

    def body(agg_ref, x1p_ref, rw, b, pw, pb, out_ref):
        x1p = x1p_ref[...]
        cnt = x1p[:, 16:17]
        mean = (agg_ref[0][:, 0:32] + agg_ref[1][:, 0:32]) / jnp.clip(cnt, 1.0, None)
        x2 = mean + _dot(x1p[:, 0:16], rw[...]) + b[...]
        out_ref[...] = _dot(x2, pw[...]) + pb[...]

    b = p['bias'].reshape(1, -1)
    pb = proj['b'].reshape(1, -1)
    return pl.pallas_call(
        body,
        grid=(grid,),
        in_specs=[
            pl.BlockSpec((2, bn, 128), lambda i: (0, i, 0)),
            pl.BlockSpec((bn, 128), lambda i: (i, 0)),
            _full_spec(p['root_w'].shape),
            _full_spec(b.shape),
            _full_spec(proj['w'].shape),
            _full_spec(pb.shape),
        ],
        out_specs=pl.BlockSpec((bn, 256), lambda i: (i, 0)),
        out_shape=jax.ShapeDtypeStruct((n, 256), jnp.float32),
    )(agg, x1pad, p['root_w'], b, proj['w'], pb)


def _tc_qkv(x, p, n):
    bn = 1000
    grid = n // bn

    def body(x_ref, wq, bq, wk, bk, wv, bv, wet,
             q0_r, q1_r, k0_r, k1_r, v0_r, v1_r, u_r):
        xb = x_ref[...]
        q = _dot(xb, wq[...]) + bq[...]
        kk = _dot(xb, wk[...]) + bk[...]
        v = _dot(xb, wv[...]) + bv[...]
        q0_r[...] = q[:, 0:128]
        q1_r[...] = q[:, 128:256]
        k0_r[...] = kk[:, 0:128]
        k1_r[...] = kk[:, 128:256]
        v0_r[...] = v[:, 0:128]
        v1_r[...] = v[:, 128:256]
        u_r[...] = jnp.concatenate(
            [_dot(q, wet[...]), jnp.zeros((bn, 112), jnp.float32)], axis=1)

    bq = p['bq'].reshape(1, -1)
    bk = p['bk'].reshape(1, -1)
    bv = p['bv'].reshape(1, -1)
    wet = p['we'].T
    half = jax.ShapeDtypeStruct((n, 128), jnp.float32)
    return pl.pallas_call(
        body,
        grid=(grid,),
        in_specs=[
            pl.BlockSpec((bn, 256), lambda i: (i, 0)),
            _full_spec((256, 256)), _full_spec((1, 256)),
            _full_spec((256, 256)), _full_spec((1, 256)),
            _full_spec((256, 256)), _full_spec((1, 256)),
            _full_spec((256, 16)),
        ],
        out_specs=[pl.BlockSpec((bn, 128), lambda i: (i, 0))] * 7,
        out_shape=[half] * 7,
    )(x, p['wq'], bq, p['wk'], bk, p['wv'], bv, wet)


def _tc_pmax(ap, scale):
    e_tot = ap.shape[1]
    be = 8000
    grid = e_tot // be

    def body(ap_ref, m_ref):
        a = jnp.sum(ap_ref[0] + ap_ref[1], axis=-1) * scale
        m_ref[...] = jnp.full((8, 128), jnp.max(a), jnp.float32)

    return pl.pallas_call(
        body,
        grid=(grid,),
        in_specs=[pl.BlockSpec((2, be, 16), lambda i: (0, i, 0))],
        out_specs=pl.BlockSpec((8, 128), lambda i: (i, 0)),
        out_shape=jax.ShapeDtypeStruct((grid * 8, 128), jnp.float32),
    )(ap)


def _tc_exp_rows(ap, pmax, ea, scale):
    e_tot = ap.shape[1]
    be = 8000
    grid = e_tot // be

    def body(ap_ref, m_ref, ea_ref, out_ref):
        cmax = jnp.max(m_ref[...])
        a = jnp.sum(ap_ref[0] + ap_ref[1], axis=-1, keepdims=True) * scale
        ew = jnp.exp(a - cmax)
        pad = jnp.zeros((be, 15), jnp.float32)
        out_ref[...] = jnp.concatenate([ea_ref[...] * ew, ew, pad], axis=1)

    return pl.pallas_call(
        body,
        grid=(grid,),
        in_specs=[
            pl.BlockSpec((2, be, 16), lambda i: (0, i, 0)),
            _full_spec(pmax.shape),
            pl.BlockSpec((be, 16), lambda i: (i, 0)),
        ],
        out_specs=pl.BlockSpec((be, 32), lambda i: (i, 0)),
        out_shape=jax.ShapeDtypeStruct((e_tot, 32), jnp.float32),
    )(ap, pmax, ea)


def _tc_tf_combine(outv, a32, x, p, lnp, n):
    bn = 1000
    grid = n // bn

    def body(ov_ref, a32_ref, x_ref, we, ws, bs, g_ref, b_ref, out_ref):
        full = jnp.concatenate([ov_ref[0], ov_ref[1]], axis=1)
        a32 = a32_ref[0] + a32_ref[1]
        s = a32[:, 16:17]
        amat = a32[:, 0:16]
        attn = (full + _dot(amat, we[...])) / (s + 1e-16)
        y = attn + _dot(x_ref[...], ws[...]) + bs[...] + x_ref[...]
        m = jnp.mean(y, axis=1, keepdims=True)
        v = jnp.mean((y - m) ** 2, axis=1, keepdims=True)
        yn = (y - m) / jnp.sqrt(v + 1e-5) * g_ref[...] + b_ref[...]
        out_ref[...] = _gelu(yn)

    bs = p['bs'].reshape(1, -1)
    g = lnp['g'].reshape(1, -1)
    b = lnp['b'].reshape(1, -1)
    return pl.pallas_call(
        body,
        grid=(grid,),
        in_specs=[
            pl.BlockSpec((2, bn, 128), lambda i: (0, i, 0)),
            pl.BlockSpec((2, bn, 128), lambda i: (0, i, 0)),
            pl.BlockSpec((bn, 256), lambda i: (i, 0)),
            _full_spec((16, 256)),
            _full_spec((256, 256)),
            _full_spec((1, 256)),
            _full_spec((1, 256)),
            _full_spec((1, 256)),
        ],
        out_specs=pl.BlockSpec((bn, 256), lambda i: (i, 0)),
        out_shape=jax.ShapeDtypeStruct((n, 256), jnp.float32),
    )(outv, a32, x, p['we'], p['ws'], bs, g, b)


def _tc_pool_head(x, head, n):
    def body(x_ref, hw, hb, out_ref):
        pooled = jnp.mean(x_ref[...], axis=0, keepdims=True)
        out_ref[...] = _dot(pooled, hw[...]) + hb[...]

    hb = head['b'].reshape(1, 1)
    return pl.pallas_call(
        body,
        grid=(1,),
        in_specs=[
            _full_spec((n, 256)),
            _full_spec((256, 1)),
            _full_spec((1, 1)),
        ],
        out_specs=_full_spec((1, 1)),
        out_shape=jax.ShapeDtypeStruct((1, 1), jnp.float32),
    )(x, head['w'], hb)


def kernel(x, edge_index, edge_attr, params):
    n = x.shape[0]
    npad = ((n + 1023) // 1024) * 1024
    e_tot = edge_index.shape[1]
    src = edge_index[0]
    dst = edge_index[1]
    gnn = params['gnn']

    e_pad = ((e_tot + 4095) // 4096) * 4096 - e_tot
    eap = jnp.pad(edge_attr, ((0, e_pad), (0, 0)))
    dstp = jnp.pad(dst, (0, e_pad))

    xpad = jnp.pad(x, ((0, 0), (0, 112)))
    xs1 = _sc_gather16(xpad, src)
    msg1 = _tc_nnconv_edge(edge_attr, xs1, gnn[0], 16, 16, with_count=True)
    agg1 = _sc_scatter32(jnp.pad(msg1, ((0, e_pad), (0, 0))), dstp, npad)
    x1pad = _tc_combine1(agg1, x, gnn[0], n)

    xs2 = _sc_gather16(x1pad, src)
    msg2 = _tc_nnconv_edge(edge_attr, xs2, gnn[1], 16, 32, with_count=False)
    agg2 = _sc_scatter32(jnp.pad(msg2, ((0, e_pad), (0, 0))), dstp, npad)
    xp = _tc_combine2_proj(agg2, x1pad, gnn[1], params['proj'], n)

    for i in range(2):
        tfp = params['tf'][i]
        q0, q1, k0, k1, v0, v1, u = _tc_qkv(xp, tfp, n)
        ap = _sc_alpha(q0, q1, k0, k1, u, edge_attr, src, dst)
        ap2 = ap.reshape(2, e_tot, 16)
        pmax = _tc_pmax(ap2, 1.0 / 16.0)
        rows32 = _tc_exp_rows(ap2, pmax, edge_attr, 1.0 / 16.0)
        outv = _sc_aggregate(v0, v1, rows32, src, dst, npad)
        a32 = _sc_scatter32(jnp.pad(rows32, ((0, e_pad), (0, 0))), dstp, npad)
        xp = _tc_tf_combine(outv, a32, xp, tfp, params['ln'][i], n)

    return _tc_pool_head(xp, params['head'], n)
